# Initial kernel scaffold; baseline (speedup 1.0000x reference)
import jax
import jax.numpy as jnp
from jax.experimental import pallas as pl


def kernel(w1, Y):
    raise NotImplementedError("write your pallas kernel here")


# speedup vs baseline: 3.3776x; 3.3776x over previous
import functools

import jax
import jax.numpy as jnp
from jax import lax
from jax.experimental import pallas as pl
from jax.experimental.pallas import tpu as pltpu
from jax.experimental.pallas import tpu_sc as plsc

N = 16384
D = 512
CP = 1024
NC, NS, L = 2, 16, 16
NW = NC * NS
RPW = N // NW
W = 128
GPR = D // W
SEGS = N * GPR
SPW = RPW * GPR
BLK = 128
NBLK = SPW // BLK
TROWS = CP * GPR
TSL = TROWS // NS

_mesh = plsc.VectorSubcoreMesh(core_axis_name="c", subcore_axis_name="s")


@functools.partial(
    pl.kernel,
    out_type=(
        jax.ShapeDtypeStruct((NC * TROWS, W), jnp.float32),
        jax.ShapeDtypeStruct((NW * L,), jnp.float32),
    ),
    mesh=_mesh,
    compiler_params=pltpu.CompilerParams(needs_layout_passes=False),
    scratch_types=[
        pltpu.VMEM_SHARED((TROWS, W), jnp.float32),
        pltpu.VMEM((BLK, W), jnp.float32),
        pltpu.VMEM((BLK, W), jnp.float32),
        pltpu.VMEM((BLK,), jnp.int32),
        pltpu.VMEM((BLK,), jnp.int32),
        pltpu.VMEM((RPW,), jnp.int32),
        pltpu.VMEM((L,), jnp.float32),
        pltpu.SemaphoreType.DMA,
        pltpu.SemaphoreType.DMA,
    ],
)
def _sc_part(wseg_hbm, y_hbm, sums_out, sq_out,
             sums_sh, buf_a, buf_b, idx_a, idx_b, y_v, sq_v, sem_a, sem_b):
    cid = lax.axis_index("c")
    sid = lax.axis_index("s")
    wid = cid * NS + sid

    def _zrow(r, _):
        def _zcol(c2, _):
            buf_a[r, pl.ds(c2 * L, L)] = jnp.zeros((L,), jnp.float32)
            return 0
        return lax.fori_loop(0, W // L, _zcol, 0)
    lax.fori_loop(0, BLK, _zrow, 0)

    pltpu.sync_copy(buf_a, sums_sh.at[pl.ds(sid * TSL, BLK)])
    pltpu.sync_copy(buf_a, sums_sh.at[pl.ds(sid * TSL + BLK, BLK)])

    pltpu.sync_copy(y_hbm.at[pl.ds(wid * RPW, RPW)], y_v)

    seg0 = wid * SPW
    bufs = (buf_a, buf_b)
    idxs = (idx_a, idx_b)
    sems = (sem_a, sem_b)
    descs = [
        pltpu.async_copy(wseg_hbm.at[pl.ds(seg0, BLK)], buf_a, sem_a),
        pltpu.async_copy(wseg_hbm.at[pl.ds(seg0 + BLK, BLK)], buf_b, sem_b),
    ]

    plsc.subcore_barrier()

    lanes = lax.iota(jnp.int32, L)
    lane_row = lanes >> 2
    lane_seg = lanes & (GPR - 1)

    accs = [jnp.zeros((L,), jnp.float32) for _ in range(W // L)]
    for s in range(NBLK):
        p = s % 2
        buf, idx, sem = bufs[p], idxs[p], sems[p]
        descs[p].wait()

        base_row = s * (BLK // GPR)
        for v in range(BLK // L):
            rows = base_row + v * (L // GPR) + lane_row
            yv = plsc.load_gather(y_v, [rows])
            idx[pl.ds(v * L, L)] = yv * GPR + lane_seg

        pltpu.sync_copy(buf, sums_sh.at[idx], add=True)

        def _row(r, a):
            new = []
            for j in range(W // L):
                x = buf[r, pl.ds(j * L, L)]
                new.append(a[j] + x * x)
            return tuple(new)
        accs = list(lax.fori_loop(0, BLK, _row, tuple(accs)))

        if s + 2 < NBLK:
            descs[p] = pltpu.async_copy(
                wseg_hbm.at[pl.ds(seg0 + (s + 2) * BLK, BLK)], buf, sem)

    acc = accs[0]
    for j in range(1, W // L):
        acc = acc + accs[j]
    sq_v[...] = acc
    pltpu.sync_copy(sq_v, sq_out.at[pl.ds(wid * L, L)])

    plsc.subcore_barrier()
    pltpu.sync_copy(sums_sh.at[pl.ds(sid * TSL, TSL)],
                    sums_out.at[pl.ds(cid * TROWS + sid * TSL, TSL)])


def _combine_body(ps_ref, y_ref, sq_ref, out_ref):
    s = ps_ref[0:TROWS, :] + ps_ref[TROWS:2 * TROWS, :]
    sq = jnp.sum(jnp.reshape(s * s, (CP, GPR * W)), axis=1)

    ids = lax.broadcasted_iota(jnp.int32, (CP, 1), 0)

    def _hist(nb, acc):
        yb = y_ref[pl.ds(nb, 1), :]
        m = (ids == yb).astype(jnp.float32)
        return acc + jnp.sum(m, axis=1)

    cnt = lax.fori_loop(0, N // CP, _hist, jnp.zeros((CP,), jnp.float32))

    tot = jnp.sum(sq_ref[...])
    val = (tot - jnp.sum(sq / jnp.maximum(cnt, 1.0))) / N
    out_ref[...] = jnp.reshape(val, (1, 1))


_combine = pl.pallas_call(
    _combine_body,
    out_shape=jax.ShapeDtypeStruct((1, 1), jnp.float32),
)


def kernel(w1, Y):
    wseg = w1.reshape(SEGS, W)
    psums, psq = _sc_part(wseg, Y)
    out = _combine(psums, Y.reshape(N // CP, CP), psq.reshape(4, 128))
    return out[0, 0]


# speedup vs baseline: 4.7567x; 1.4083x over previous
import functools

import jax
import jax.numpy as jnp
from jax import lax
from jax.experimental import pallas as pl
from jax.experimental.pallas import tpu as pltpu
from jax.experimental.pallas import tpu_sc as plsc

N = 16384
D = 512
CP = 1024
NC, NS, L = 2, 16, 16
NW = NC * NS
RPW = N // NW
W = 128
GPR = D // W
SEGS = N * GPR
SPW = RPW * GPR
BLK = 128
NBLK = SPW // BLK
TROWS = CP * GPR
TSL = TROWS // NS

_mesh = plsc.VectorSubcoreMesh(core_axis_name="c", subcore_axis_name="s")


@functools.partial(
    pl.kernel,
    out_type=(
        jax.ShapeDtypeStruct((NC * TROWS, W), jnp.float32),
        jax.ShapeDtypeStruct((NW * L,), jnp.float32),
    ),
    mesh=_mesh,
    compiler_params=pltpu.CompilerParams(needs_layout_passes=False),
    scratch_types=[
        pltpu.VMEM_SHARED((TROWS, W), jnp.float32),
        pltpu.VMEM((BLK, W), jnp.float32),
        pltpu.VMEM((BLK, W), jnp.float32),
        pltpu.VMEM((BLK,), jnp.int32),
        pltpu.VMEM((BLK,), jnp.int32),
        pltpu.VMEM((RPW,), jnp.int32),
        pltpu.VMEM((L,), jnp.float32),
        pltpu.SemaphoreType.DMA,
        pltpu.SemaphoreType.DMA,
    ],
)
def _sc_part(w_hbm, y_hbm, sums_out, sq_out,
             sums_sh, buf_a, buf_b, idx_a, idx_b, y_v, sq_v, sem_a, sem_b):
    cid = lax.axis_index("c")
    sid = lax.axis_index("s")
    wid = cid * NS + sid

    def _zrow(r, _):
        def _zcol(c2, _):
            buf_a[r, pl.ds(c2 * L, L)] = jnp.zeros((L,), jnp.float32)
            return 0
        return lax.fori_loop(0, W // L, _zcol, 0)
    lax.fori_loop(0, BLK, _zrow, 0)

    pltpu.sync_copy(buf_a, sums_sh.at[pl.ds(sid * TSL, BLK)])
    pltpu.sync_copy(buf_a, sums_sh.at[pl.ds(sid * TSL + BLK, BLK)])

    pltpu.sync_copy(y_hbm.at[pl.ds(wid * RPW, RPW)], y_v)

    RPB = BLK // GPR
    row0 = wid * RPW
    bufs = (buf_a, buf_b)
    idxs = (idx_a, idx_b)
    sems = (sem_a, sem_b)

    def _start_load(s, buf, sem):
        return [
            pltpu.async_copy(
                w_hbm.at[pl.ds(row0 + s * RPB, RPB), pl.ds(g * W, W)],
                buf.at[pl.ds(g * RPB, RPB)], sem)
            for g in range(GPR)
        ]

    descs = [_start_load(0, buf_a, sem_a), _start_load(1, buf_b, sem_b)]

    plsc.subcore_barrier()

    accs = [jnp.zeros((L,), jnp.float32) for _ in range(W // L)]
    for s in range(NBLK):
        p = s % 2
        buf, idx, sem = bufs[p], idxs[p], sems[p]
        for dsc in descs[p]:
            dsc.wait()

        for v in range(BLK // L):
            yv = y_v[pl.ds(s * RPB + (v & 1) * L, L)]
            idx[pl.ds(v * L, L)] = yv * GPR + (v >> 1)

        pltpu.sync_copy(buf, sums_sh.at[idx], add=True)

        def _row(r, a):
            new = []
            for j in range(W // L):
                x = buf[r, pl.ds(j * L, L)]
                new.append(a[j] + x * x)
            return tuple(new)
        accs = list(lax.fori_loop(0, BLK, _row, tuple(accs)))

        if s + 2 < NBLK:
            descs[p] = _start_load(s + 2, buf, sem)

    acc = accs[0]
    for j in range(1, W // L):
        acc = acc + accs[j]
    sq_v[...] = acc
    pltpu.sync_copy(sq_v, sq_out.at[pl.ds(wid * L, L)])

    plsc.subcore_barrier()
    pltpu.sync_copy(sums_sh.at[pl.ds(sid * TSL, TSL)],
                    sums_out.at[pl.ds(cid * TROWS + sid * TSL, TSL)])


def _combine_body(ps_ref, y_ref, sq_ref, out_ref):
    s = ps_ref[0:TROWS, :] + ps_ref[TROWS:2 * TROWS, :]
    sq = jnp.sum(jnp.reshape(s * s, (CP, GPR * W)), axis=1)

    ids = lax.broadcasted_iota(jnp.int32, (CP, 1), 0)

    def _hist(nb, acc):
        yb = y_ref[pl.ds(nb, 1), :]
        m = (ids == yb).astype(jnp.float32)
        return acc + jnp.sum(m, axis=1)

    cnt = lax.fori_loop(0, N // CP, _hist, jnp.zeros((CP,), jnp.float32))

    tot = jnp.sum(sq_ref[...])
    val = (tot - jnp.sum(sq / jnp.maximum(cnt, 1.0))) / N
    out_ref[...] = jnp.reshape(val, (1, 1))


_combine = pl.pallas_call(
    _combine_body,
    out_shape=jax.ShapeDtypeStruct((1, 1), jnp.float32),
)


def kernel(w1, Y):
    psums, psq = _sc_part(w1, Y)
    out = _combine(psums, Y.reshape(N // CP, CP), psq.reshape(4, 128))
    return out[0, 0]


# speedup vs baseline: 5.0007x; 1.0513x over previous
import functools

import jax
import jax.numpy as jnp
from jax import lax
from jax.experimental import pallas as pl
from jax.experimental.pallas import tpu as pltpu
from jax.experimental.pallas import tpu_sc as plsc

N = 16384
D = 512
CP = 1024
NC, NS, L = 2, 16, 16
NW = NC * NS
RPW = N // NW
W = 128
GPR = D // W
SEGS = N * GPR
SPW = RPW * GPR
BLK = 128
NBLK = SPW // BLK
TROWS = CP * GPR
TSL = TROWS // NS

_mesh = plsc.VectorSubcoreMesh(core_axis_name="c", subcore_axis_name="s")


@functools.partial(
    pl.kernel,
    out_type=jax.ShapeDtypeStruct((NC * TROWS, W), jnp.float32),
    mesh=_mesh,
    compiler_params=pltpu.CompilerParams(needs_layout_passes=False),
    scratch_types=[
        pltpu.VMEM_SHARED((TROWS, W), jnp.float32),
        pltpu.VMEM((BLK, W), jnp.float32),
        pltpu.VMEM((BLK, W), jnp.float32),
        pltpu.VMEM((BLK,), jnp.int32),
        pltpu.VMEM((BLK,), jnp.int32),
        pltpu.VMEM((RPW,), jnp.int32),
        pltpu.SemaphoreType.DMA,
        pltpu.SemaphoreType.DMA,
    ],
)
def _sc_part(w_hbm, y_hbm, sums_out,
             sums_sh, buf_a, buf_b, idx_a, idx_b, y_v, sem_a, sem_b):
    cid = lax.axis_index("c")
    sid = lax.axis_index("s")
    wid = cid * NS + sid

    def _zrow(r, _):
        def _zcol(c2, _):
            buf_a[r, pl.ds(c2 * L, L)] = jnp.zeros((L,), jnp.float32)
            return 0
        return lax.fori_loop(0, W // L, _zcol, 0)
    lax.fori_loop(0, BLK, _zrow, 0)

    pltpu.sync_copy(buf_a, sums_sh.at[pl.ds(sid * TSL, BLK)])
    pltpu.sync_copy(buf_a, sums_sh.at[pl.ds(sid * TSL + BLK, BLK)])

    pltpu.sync_copy(y_hbm.at[pl.ds(wid * RPW, RPW)], y_v)

    RPB = BLK // GPR
    row0 = wid * RPW
    bufs = (buf_a, buf_b)
    idxs = (idx_a, idx_b)
    sems = (sem_a, sem_b)

    def _start_load(s, buf, sem):
        return [
            pltpu.async_copy(
                w_hbm.at[pl.ds(row0 + s * RPB, RPB), pl.ds(g * W, W)],
                buf.at[pl.ds(g * RPB, RPB)], sem)
            for g in range(GPR)
        ]

    descs = [_start_load(0, buf_a, sem_a), _start_load(1, buf_b, sem_b)]

    plsc.subcore_barrier()

    for s in range(NBLK):
        p = s % 2
        buf, idx, sem = bufs[p], idxs[p], sems[p]
        for dsc in descs[p]:
            dsc.wait()

        for v in range(BLK // L):
            yv = y_v[pl.ds(s * RPB + (v & 1) * L, L)]
            idx[pl.ds(v * L, L)] = yv * GPR + (v >> 1)

        pltpu.sync_copy(buf, sums_sh.at[idx], add=True)

        if s + 2 < NBLK:
            descs[p] = _start_load(s + 2, buf, sem)

    plsc.subcore_barrier()
    pltpu.sync_copy(sums_sh.at[pl.ds(sid * TSL, TSL)],
                    sums_out.at[pl.ds(cid * TROWS + sid * TSL, TSL)])


def _sumsq_body(w_ref, out_ref):
    @pl.when(pl.program_id(0) == 0)
    def _():
        out_ref[...] = jnp.zeros((8, 512), jnp.float32)

    x = w_ref[...]
    out_ref[...] += jnp.sum(jnp.reshape(x * x, (128, 8, 512)), axis=0)


_sumsq = pl.pallas_call(
    _sumsq_body,
    grid=(16,),
    in_specs=[pl.BlockSpec((1024, 512), lambda i: (i, 0))],
    out_specs=pl.BlockSpec((8, 512), lambda i: (0, 0)),
    out_shape=jax.ShapeDtypeStruct((8, 512), jnp.float32),
)


def _combine_body(ps_ref, y_ref, sq_ref, out_ref):
    s = ps_ref[0:TROWS, :] + ps_ref[TROWS:2 * TROWS, :]
    sq = jnp.sum(jnp.reshape(s * s, (CP, GPR * W)), axis=1)

    ids = lax.broadcasted_iota(jnp.int32, (CP, 1), 0)

    def _hist(nb, acc):
        yb = y_ref[pl.ds(nb, 1), :]
        m = (ids == yb).astype(jnp.float32)
        return acc + jnp.sum(m, axis=1)

    cnt = lax.fori_loop(0, N // CP, _hist, jnp.zeros((CP,), jnp.float32))

    tot = jnp.sum(sq_ref[...])
    val = (tot - jnp.sum(sq / jnp.maximum(cnt, 1.0))) / N
    out_ref[...] = jnp.reshape(val, (1, 1))


_combine = pl.pallas_call(
    _combine_body,
    out_shape=jax.ShapeDtypeStruct((1, 1), jnp.float32),
)


def kernel(w1, Y):
    psums = _sc_part(w1, Y)
    psq = _sumsq(w1)
    out = _combine(psums, Y.reshape(N // CP, CP), psq)
    return out[0, 0]
